# baseline (speedup 1.0000x reference)
import functools

import jax
import jax.numpy as jnp
from jax.experimental import pallas as pl
import jax.experimental.pallas.tpu as pltpu

N_ROWS = 8192
N_CODES = 8192
DIM = 256
BLOCK_ROWS = 256
N_BLOCKS = N_ROWS // BLOCK_ROWS


def _vq_kernel(x_ref, xn_ref, en_ref, emb_ref, et_ref,
               q_ref, cb_ref, cm_ref, pp_ref,
               counts_ref, loss_ref):
    i = pl.program_id(0)

    @pl.when(i == 0)
    def _init():
        counts_ref[...] = jnp.zeros_like(counts_ref)
        loss_ref[0, 0] = 0.0

    x = x_ref[...]
    mm = jax.lax.dot_general(
        x, et_ref[...],
        dimension_numbers=(((1,), (0,)), ((), ())),
        preferred_element_type=jnp.float32)
    d2 = (xn_ref[...] + en_ref[...]) - 2.0 * mm
    dist = jnp.maximum(d2, 0.0)
    minval = jnp.min(dist, axis=1, keepdims=True)
    jcol = jax.lax.broadcasted_iota(jnp.int32, dist.shape, 1)
    idx = jnp.min(jnp.where(dist == minval, jcol, N_CODES),
                  axis=1, keepdims=True)
    one_hot = (jcol == idx).astype(jnp.float32)
    q = jax.lax.dot_general(
        one_hot, emb_ref[...],
        dimension_numbers=(((1,), (0,)), ((), ())),
        preferred_element_type=jnp.float32,
        precision=jax.lax.Precision.HIGHEST)
    q_ref[...] = x + (q - x)
    counts_ref[...] += jnp.sum(one_hot, axis=0, keepdims=True)
    loss_ref[0, 0] += jnp.sum(minval)

    @pl.when(i == N_BLOCKS - 1)
    def _finish():
        total = loss_ref[0, 0]
        mean_sq = total / (N_ROWS * DIM)
        cb_ref[...] = jnp.reshape(mean_sq, (1, 1))
        cm_ref[...] = jnp.reshape(0.25 * mean_sq, (1, 1))
        p = counts_ref[...] * (1.0 / N_ROWS)
        ent = jnp.sum(p * jnp.log(p + 1e-10))
        pp_ref[...] = jnp.reshape(jnp.exp(-ent), (1, 1))


@jax.jit
def kernel(x, embedding):
    x_flat = x.reshape(-1, DIM)
    xn = jnp.sum(x_flat ** 2, axis=1, keepdims=True)
    en = jnp.sum(embedding ** 2, axis=1)[None, :]
    et = embedding.T

    grid = (N_BLOCKS,)
    q, cb, cm, pp = pl.pallas_call(
        _vq_kernel,
        grid=grid,
        in_specs=[
            pl.BlockSpec((BLOCK_ROWS, DIM), lambda i: (i, 0)),
            pl.BlockSpec((BLOCK_ROWS, 1), lambda i: (i, 0)),
            pl.BlockSpec((1, N_CODES), lambda i: (0, 0)),
            pl.BlockSpec((N_CODES, DIM), lambda i: (0, 0)),
            pl.BlockSpec((DIM, N_CODES), lambda i: (0, 0)),
        ],
        out_specs=[
            pl.BlockSpec((BLOCK_ROWS, DIM), lambda i: (i, 0)),
            pl.BlockSpec((1, 1), lambda i: (0, 0)),
            pl.BlockSpec((1, 1), lambda i: (0, 0)),
            pl.BlockSpec((1, 1), lambda i: (0, 0)),
        ],
        out_shape=[
            jax.ShapeDtypeStruct((N_ROWS, DIM), jnp.float32),
            jax.ShapeDtypeStruct((1, 1), jnp.float32),
            jax.ShapeDtypeStruct((1, 1), jnp.float32),
            jax.ShapeDtypeStruct((1, 1), jnp.float32),
        ],
        scratch_shapes=[
            pltpu.VMEM((1, N_CODES), jnp.float32),
            pltpu.SMEM((1, 1), jnp.float32),
        ],
    )(x_flat, xn, en, embedding, et)

    quantized_st = q.reshape(x.shape)
    return (quantized_st, cm.reshape(()), cb.reshape(()), pp.reshape(()))


# speedup vs baseline: 1.7652x; 1.7652x over previous
import functools

import jax
import jax.numpy as jnp
from jax.experimental import pallas as pl
import jax.experimental.pallas.tpu as pltpu

N_ROWS = 8192
N_CODES = 8192
DIM = 256
BLOCK_ROWS = 256
N_BLOCKS = N_ROWS // BLOCK_ROWS


def _vq_kernel(x_ref, xn_ref, en_ref, emb_ref, et_ref,
               q_ref, cb_ref, cm_ref, pp_ref,
               counts_ref, loss_ref):
    i = pl.program_id(0)

    @pl.when(i == 0)
    def _init():
        counts_ref[...] = jnp.zeros_like(counts_ref)
        loss_ref[0, 0] = 0.0

    x = x_ref[...]
    mm = jax.lax.dot_general(
        x, et_ref[...],
        dimension_numbers=(((1,), (0,)), ((), ())),
        preferred_element_type=jnp.float32)
    d2 = (xn_ref[...] + en_ref[...]) - 2.0 * mm
    dist = jnp.maximum(d2, 0.0)
    minval = jnp.min(dist, axis=1, keepdims=True)
    jcol = jax.lax.broadcasted_iota(jnp.int32, dist.shape, 1)
    idx = jnp.min(jnp.where(dist == minval, jcol, N_CODES),
                  axis=1, keepdims=True)
    one_hot = (jcol == idx).astype(jnp.float32)
    q = jax.lax.dot_general(
        one_hot, emb_ref[...],
        dimension_numbers=(((1,), (0,)), ((), ())),
        preferred_element_type=jnp.float32)
    q_ref[...] = x + (q - x)
    counts_ref[...] += jnp.sum(one_hot, axis=0, keepdims=True)
    loss_ref[0, 0] += jnp.sum(minval)

    @pl.when(i == N_BLOCKS - 1)
    def _finish():
        total = loss_ref[0, 0]
        mean_sq = total / (N_ROWS * DIM)
        cb_ref[...] = jnp.reshape(mean_sq, (1, 1))
        cm_ref[...] = jnp.reshape(0.25 * mean_sq, (1, 1))
        p = counts_ref[...] * (1.0 / N_ROWS)
        ent = jnp.sum(p * jnp.log(p + 1e-10))
        pp_ref[...] = jnp.reshape(jnp.exp(-ent), (1, 1))


@jax.jit
def kernel(x, embedding):
    x_flat = x.reshape(-1, DIM)
    xn = jnp.sum(x_flat ** 2, axis=1, keepdims=True)
    en = jnp.sum(embedding ** 2, axis=1)[None, :]
    et = embedding.T

    grid = (N_BLOCKS,)
    q, cb, cm, pp = pl.pallas_call(
        _vq_kernel,
        grid=grid,
        in_specs=[
            pl.BlockSpec((BLOCK_ROWS, DIM), lambda i: (i, 0)),
            pl.BlockSpec((BLOCK_ROWS, 1), lambda i: (i, 0)),
            pl.BlockSpec((1, N_CODES), lambda i: (0, 0)),
            pl.BlockSpec((N_CODES, DIM), lambda i: (0, 0)),
            pl.BlockSpec((DIM, N_CODES), lambda i: (0, 0)),
        ],
        out_specs=[
            pl.BlockSpec((BLOCK_ROWS, DIM), lambda i: (i, 0)),
            pl.BlockSpec((1, 1), lambda i: (0, 0)),
            pl.BlockSpec((1, 1), lambda i: (0, 0)),
            pl.BlockSpec((1, 1), lambda i: (0, 0)),
        ],
        out_shape=[
            jax.ShapeDtypeStruct((N_ROWS, DIM), jnp.float32),
            jax.ShapeDtypeStruct((1, 1), jnp.float32),
            jax.ShapeDtypeStruct((1, 1), jnp.float32),
            jax.ShapeDtypeStruct((1, 1), jnp.float32),
        ],
        scratch_shapes=[
            pltpu.VMEM((1, N_CODES), jnp.float32),
            pltpu.SMEM((1, 1), jnp.float32),
        ],
    )(x_flat, xn, en, embedding, et)

    quantized_st = q.reshape(x.shape)
    return (quantized_st, cm.reshape(()), cb.reshape(()), pp.reshape(()))


# speedup vs baseline: 1.9650x; 1.1132x over previous
import functools

import jax
import jax.numpy as jnp
from jax import lax
from jax.experimental import pallas as pl
import jax.experimental.pallas.tpu as pltpu
from jax.experimental.pallas import tpu_sc as plsc

N_ROWS = 8192
N_CODES = 8192
DIM = 256
BLOCK_ROWS = 256
N_BLOCKS = N_ROWS // BLOCK_ROWS

SC_CORES = 2
SC_SUBCORES = 16
SC_TILES = SC_CORES * SC_SUBCORES
ROWS_PER_TILE = N_ROWS // SC_TILES
IDX_CHUNK = 128
N_CHUNKS = ROWS_PER_TILE // IDX_CHUNK


def _argmin_kernel(x_ref, xn_ref, en_ref, et_ref, jcol_ref,
                   idx_ref, cb_ref, cm_ref, pp_ref,
                   counts_ref, loss_ref):
    i = pl.program_id(0)

    @pl.when(i == 0)
    def _init():
        counts_ref[...] = jnp.zeros_like(counts_ref)
        loss_ref[0, 0] = 0.0

    x = x_ref[...]
    mm = jax.lax.dot_general(
        x, et_ref[...],
        dimension_numbers=(((1,), (0,)), ((), ())),
        preferred_element_type=jnp.float32)
    d2 = (xn_ref[...] + en_ref[...]) - 2.0 * mm
    dist = jnp.maximum(d2, 0.0)
    minval = jnp.min(dist, axis=1, keepdims=True)
    jcol = jcol_ref[...]
    idx = jnp.min(jnp.where(dist == minval, jcol, N_CODES),
                  axis=1, keepdims=True)
    idx_ref[...] = idx
    counts_ref[...] += jnp.sum((jcol == idx).astype(jnp.float32),
                               axis=0, keepdims=True)
    loss_ref[0, 0] += jnp.sum(minval)

    @pl.when(i == N_BLOCKS - 1)
    def _finish():
        mean_sq = loss_ref[0, 0] / (N_ROWS * DIM)
        cb_ref[...] = jnp.reshape(mean_sq, (1, 1))
        cm_ref[...] = jnp.reshape(0.25 * mean_sq, (1, 1))
        p = counts_ref[...] * (1.0 / N_ROWS)
        ent = jnp.sum(p * jnp.log(p + 1e-10))
        pp_ref[...] = jnp.reshape(jnp.exp(-ent), (1, 1))


@functools.partial(
    pl.kernel,
    mesh=plsc.VectorSubcoreMesh(core_axis_name="c", subcore_axis_name="s"),
    out_type=jax.ShapeDtypeStruct((N_ROWS, DIM), jnp.float32),
    scratch_types=[
        pltpu.VMEM((N_CHUNKS, IDX_CHUNK), jnp.int32),
        pltpu.VMEM((ROWS_PER_TILE, DIM), jnp.float32),
        pltpu.SemaphoreType.DMA,
    ],
)
def _sc_gather(table_hbm, idx_hbm, q_hbm, idx_v, rows_v, sem):
    cid = lax.axis_index("c")
    sid = lax.axis_index("s")
    wid = sid * SC_CORES + cid
    base = wid * ROWS_PER_TILE

    pltpu.sync_copy(idx_hbm.at[wid], idx_v)

    for j in range(N_CHUNKS):
        pltpu.async_copy(table_hbm.at[idx_v.at[j]],
                         rows_v.at[pl.ds(j * IDX_CHUNK, IDX_CHUNK)],
                         sem).wait()

    pltpu.sync_copy(rows_v, q_hbm.at[pl.ds(base, ROWS_PER_TILE)])


def _st_kernel(x_ref, q_ref, out_ref):
    x = x_ref[...]
    q = q_ref[...]
    out_ref[...] = x + (q - x)


@jax.jit
def kernel(x, embedding):
    x_flat = x.reshape(-1, DIM)
    xn = jnp.sum(x_flat ** 2, axis=1, keepdims=True)
    en = jnp.sum(embedding ** 2, axis=1)[None, :]
    et = embedding.T
    jcol = jax.lax.broadcasted_iota(jnp.int32, (1, N_CODES), 1)

    idx, cb, cm, pp = pl.pallas_call(
        _argmin_kernel,
        grid=(N_BLOCKS,),
        in_specs=[
            pl.BlockSpec((BLOCK_ROWS, DIM), lambda i: (i, 0)),
            pl.BlockSpec((BLOCK_ROWS, 1), lambda i: (i, 0)),
            pl.BlockSpec((1, N_CODES), lambda i: (0, 0)),
            pl.BlockSpec((DIM, N_CODES), lambda i: (0, 0)),
            pl.BlockSpec((1, N_CODES), lambda i: (0, 0)),
        ],
        out_specs=[
            pl.BlockSpec((BLOCK_ROWS, 1), lambda i: (i, 0)),
            pl.BlockSpec((1, 1), lambda i: (0, 0)),
            pl.BlockSpec((1, 1), lambda i: (0, 0)),
            pl.BlockSpec((1, 1), lambda i: (0, 0)),
        ],
        out_shape=[
            jax.ShapeDtypeStruct((N_ROWS, 1), jnp.int32),
            jax.ShapeDtypeStruct((1, 1), jnp.float32),
            jax.ShapeDtypeStruct((1, 1), jnp.float32),
            jax.ShapeDtypeStruct((1, 1), jnp.float32),
        ],
        scratch_shapes=[
            pltpu.VMEM((1, N_CODES), jnp.float32),
            pltpu.SMEM((1, 1), jnp.float32),
        ],
    )(x_flat, xn, en, et, jcol)

    idx3 = idx.reshape(SC_TILES, N_CHUNKS, IDX_CHUNK)
    q = _sc_gather(embedding, idx3)

    q_st = pl.pallas_call(
        _st_kernel,
        grid=(N_BLOCKS,),
        in_specs=[
            pl.BlockSpec((BLOCK_ROWS, DIM), lambda i: (i, 0)),
            pl.BlockSpec((BLOCK_ROWS, DIM), lambda i: (i, 0)),
        ],
        out_specs=pl.BlockSpec((BLOCK_ROWS, DIM), lambda i: (i, 0)),
        out_shape=jax.ShapeDtypeStruct((N_ROWS, DIM), jnp.float32),
    )(x_flat, q)

    quantized_st = q_st.reshape(x.shape)
    return (quantized_st, cm.reshape(()), cb.reshape(()), pp.reshape(()))


# speedup vs baseline: 2.2401x; 1.1400x over previous
import functools

import jax
import jax.numpy as jnp
from jax import lax
from jax.experimental import pallas as pl
import jax.experimental.pallas.tpu as pltpu
from jax.experimental.pallas import tpu_sc as plsc

N_ROWS = 8192
N_CODES = 8192
DIM = 256
BLOCK_ROWS = 512
N_BLOCKS = N_ROWS // BLOCK_ROWS

SC_CORES = 2
SC_SUBCORES = 16
SC_TILES = SC_CORES * SC_SUBCORES
ROWS_PER_TILE = N_ROWS // SC_TILES
IDX_CHUNK = 128
N_CHUNKS = ROWS_PER_TILE // IDX_CHUNK


def _argmin_kernel(x_ref, xn_ref, en_ref, et_ref, jcol_ref,
                   idx_ref, cb_ref, cm_ref, pp_ref,
                   counts_ref, loss_ref):
    i = pl.program_id(0)

    @pl.when(i == 0)
    def _init():
        counts_ref[...] = jnp.zeros_like(counts_ref)
        loss_ref[0, 0] = 0.0

    x = x_ref[...]
    mm = jax.lax.dot_general(
        x, et_ref[...],
        dimension_numbers=(((1,), (0,)), ((), ())),
        preferred_element_type=jnp.float32)
    d2 = (xn_ref[...] + en_ref[...]) - 2.0 * mm
    dist = jnp.maximum(d2, 0.0)
    minval = jnp.min(dist, axis=1, keepdims=True)
    jcol = jcol_ref[...]
    idxf = jnp.min(jnp.where(dist == minval, jcol, float(N_CODES)),
                   axis=1, keepdims=True)
    idx_ref[...] = idxf.astype(jnp.int32)
    counts_ref[...] += jnp.sum((jcol == idxf).astype(jnp.float32),
                               axis=0, keepdims=True)
    loss_ref[0, 0] += jnp.sum(minval)

    @pl.when(i == N_BLOCKS - 1)
    def _finish():
        mean_sq = loss_ref[0, 0] / (N_ROWS * DIM)
        cb_ref[...] = jnp.reshape(mean_sq, (1, 1))
        cm_ref[...] = jnp.reshape(0.25 * mean_sq, (1, 1))
        p = counts_ref[...] * (1.0 / N_ROWS)
        ent = jnp.sum(p * jnp.log(p + 1e-10))
        pp_ref[...] = jnp.reshape(jnp.exp(-ent), (1, 1))


@functools.partial(
    pl.kernel,
    mesh=plsc.VectorSubcoreMesh(core_axis_name="c", subcore_axis_name="s"),
    out_type=jax.ShapeDtypeStruct((N_ROWS, DIM), jnp.float32),
    scratch_types=[
        pltpu.VMEM((N_CHUNKS, IDX_CHUNK), jnp.int32),
        pltpu.VMEM((ROWS_PER_TILE, DIM), jnp.float32),
        pltpu.SemaphoreType.DMA,
    ],
)
def _sc_gather(table_hbm, idx_hbm, q_hbm, idx_v, rows_v, sem):
    cid = lax.axis_index("c")
    sid = lax.axis_index("s")
    wid = sid * SC_CORES + cid
    base = wid * ROWS_PER_TILE

    pltpu.sync_copy(idx_hbm.at[wid], idx_v)

    for j in range(N_CHUNKS):
        pltpu.async_copy(table_hbm.at[idx_v.at[j]],
                         rows_v.at[pl.ds(j * IDX_CHUNK, IDX_CHUNK)],
                         sem).wait()

    pltpu.sync_copy(rows_v, q_hbm.at[pl.ds(base, ROWS_PER_TILE)])


def _st_kernel(x_ref, q_ref, out_ref):
    x = x_ref[...]
    q = q_ref[...]
    out_ref[...] = x + (q - x)


@jax.jit
def kernel(x, embedding):
    x_flat = x.reshape(-1, DIM)
    xn = jnp.sum(x_flat ** 2, axis=1, keepdims=True)
    en = jnp.sum(embedding ** 2, axis=1)[None, :]
    et = embedding.T
    jcol = jax.lax.broadcasted_iota(jnp.float32, (1, N_CODES), 1)

    idx, cb, cm, pp = pl.pallas_call(
        _argmin_kernel,
        grid=(N_BLOCKS,),
        in_specs=[
            pl.BlockSpec((BLOCK_ROWS, DIM), lambda i: (i, 0)),
            pl.BlockSpec((BLOCK_ROWS, 1), lambda i: (i, 0)),
            pl.BlockSpec((1, N_CODES), lambda i: (0, 0)),
            pl.BlockSpec((DIM, N_CODES), lambda i: (0, 0)),
            pl.BlockSpec((1, N_CODES), lambda i: (0, 0)),
        ],
        out_specs=[
            pl.BlockSpec((BLOCK_ROWS, 1), lambda i: (i, 0)),
            pl.BlockSpec((1, 1), lambda i: (0, 0)),
            pl.BlockSpec((1, 1), lambda i: (0, 0)),
            pl.BlockSpec((1, 1), lambda i: (0, 0)),
        ],
        out_shape=[
            jax.ShapeDtypeStruct((N_ROWS, 1), jnp.int32),
            jax.ShapeDtypeStruct((1, 1), jnp.float32),
            jax.ShapeDtypeStruct((1, 1), jnp.float32),
            jax.ShapeDtypeStruct((1, 1), jnp.float32),
        ],
        scratch_shapes=[
            pltpu.VMEM((1, N_CODES), jnp.float32),
            pltpu.SMEM((1, 1), jnp.float32),
        ],
    )(x_flat, xn, en, et, jcol)

    idx3 = idx.reshape(SC_TILES, N_CHUNKS, IDX_CHUNK)
    q = _sc_gather(embedding, idx3)

    q_st = pl.pallas_call(
        _st_kernel,
        grid=(N_BLOCKS,),
        in_specs=[
            pl.BlockSpec((BLOCK_ROWS, DIM), lambda i: (i, 0)),
            pl.BlockSpec((BLOCK_ROWS, DIM), lambda i: (i, 0)),
        ],
        out_specs=pl.BlockSpec((BLOCK_ROWS, DIM), lambda i: (i, 0)),
        out_shape=jax.ShapeDtypeStruct((N_ROWS, DIM), jnp.float32),
    )(x_flat, q)

    quantized_st = q_st.reshape(x.shape)
    return (quantized_st, cm.reshape(()), cb.reshape(()), pp.reshape(()))


# speedup vs baseline: 2.4800x; 1.1071x over previous
import functools

import jax
import jax.numpy as jnp
from jax import lax
from jax.experimental import pallas as pl
import jax.experimental.pallas.tpu as pltpu
from jax.experimental.pallas import tpu_sc as plsc

N_ROWS = 8192
N_CODES = 8192
DIM = 256
BLOCK_ROWS = 512
N_BLOCKS = N_ROWS // BLOCK_ROWS

SC_CORES = 2
SC_SUBCORES = 16
SC_TILES = SC_CORES * SC_SUBCORES
ROWS_PER_TILE = N_ROWS // SC_TILES
IDX_CHUNK = 128
N_CHUNKS = ROWS_PER_TILE // IDX_CHUNK
HIST_W = 16
HIST_STRIPE = N_CODES // SC_SUBCORES


def _argmin_kernel(x_ref, xn_ref, en_ref, et_ref, jcol_ref,
                   idx_ref, cb_ref, cm_ref, loss_ref):
    i = pl.program_id(0)

    @pl.when(i == 0)
    def _init():
        loss_ref[0, 0] = 0.0

    x = x_ref[...]
    mm = jax.lax.dot_general(
        x, et_ref[...],
        dimension_numbers=(((1,), (0,)), ((), ())),
        preferred_element_type=jnp.float32)
    d2 = (xn_ref[...] + en_ref[...]) - 2.0 * mm
    dist = jnp.maximum(d2, 0.0)
    minval = jnp.min(dist, axis=1, keepdims=True)
    idxf = jnp.min(jnp.where(dist == minval, jcol_ref[...], float(N_CODES)),
                   axis=1, keepdims=True)
    idx_ref[...] = idxf.astype(jnp.int32)
    loss_ref[0, 0] += jnp.sum(minval)

    @pl.when(i == N_BLOCKS - 1)
    def _finish():
        mean_sq = loss_ref[0, 0] / (N_ROWS * DIM)
        cb_ref[...] = jnp.reshape(mean_sq, (1, 1))
        cm_ref[...] = jnp.reshape(0.25 * mean_sq, (1, 1))


@functools.partial(
    pl.kernel,
    mesh=plsc.VectorSubcoreMesh(core_axis_name="c", subcore_axis_name="s"),
    out_type=[
        jax.ShapeDtypeStruct((N_ROWS, DIM), jnp.float32),
        jax.ShapeDtypeStruct((SC_CORES, N_CODES, HIST_W), jnp.float32),
    ],
    scratch_types=[
        pltpu.VMEM((N_CHUNKS, IDX_CHUNK), jnp.int32),
        pltpu.VMEM((IDX_CHUNK, DIM), jnp.float32),
        pltpu.VMEM((IDX_CHUNK, HIST_W), jnp.float32),
        pltpu.VMEM_SHARED((N_CODES, HIST_W), jnp.float32),
        pltpu.SemaphoreType.DMA,
    ],
)
def _sc_gather(table_hbm, idx_hbm, zeros_hbm, ones_hbm, q_hbm, counts_hbm,
               idx_v, rows_v, ones_v, counts_sh, sem):
    cid = lax.axis_index("c")
    sid = lax.axis_index("s")
    wid = sid * SC_CORES + cid
    base = wid * ROWS_PER_TILE
    stripe = sid * HIST_STRIPE

    pltpu.sync_copy(idx_hbm.at[wid], idx_v)
    pltpu.sync_copy(ones_hbm, ones_v)
    pltpu.sync_copy(zeros_hbm.at[pl.ds(stripe, HIST_STRIPE)],
                    counts_sh.at[pl.ds(stripe, HIST_STRIPE)])

    plsc.subcore_barrier()

    for j in range(N_CHUNKS):
        pltpu.async_copy(table_hbm.at[idx_v.at[j]], rows_v, sem).wait()
        pltpu.sync_copy(rows_v, q_hbm.at[pl.ds(base + j * IDX_CHUNK,
                                               IDX_CHUNK)])
        pltpu.sync_copy(ones_v, counts_sh.at[idx_v.at[j]], add=True)

    plsc.subcore_barrier()

    pltpu.sync_copy(counts_sh.at[pl.ds(stripe, HIST_STRIPE)],
                    counts_hbm.at[cid, pl.ds(stripe, HIST_STRIPE)])


def _st_kernel(x_ref, q_ref, c_ref, out_ref, pp_ref):
    i = pl.program_id(0)
    x = x_ref[...]
    q = q_ref[...]
    out_ref[...] = x + (q - x)

    @pl.when(i == N_BLOCKS - 1)
    def _finish():
        c = c_ref[0] + c_ref[1]
        cnt = jnp.sum(c, axis=1, keepdims=True) * (1.0 / HIST_W)
        p = cnt * (1.0 / N_ROWS)
        ent = jnp.sum(p * jnp.log(p + 1e-10))
        pp_ref[...] = jnp.reshape(jnp.exp(-ent), (1, 1))


@jax.jit
def kernel(x, embedding):
    x_flat = x.reshape(-1, DIM)
    xn = jnp.sum(x_flat ** 2, axis=1, keepdims=True)
    en = jnp.sum(embedding ** 2, axis=1)[None, :]
    et = embedding.T
    jcol = jax.lax.broadcasted_iota(jnp.float32, (1, N_CODES), 1)

    idx, cb, cm = pl.pallas_call(
        _argmin_kernel,
        grid=(N_BLOCKS,),
        in_specs=[
            pl.BlockSpec((BLOCK_ROWS, DIM), lambda i: (i, 0)),
            pl.BlockSpec((BLOCK_ROWS, 1), lambda i: (i, 0)),
            pl.BlockSpec((1, N_CODES), lambda i: (0, 0)),
            pl.BlockSpec((DIM, N_CODES), lambda i: (0, 0)),
            pl.BlockSpec((1, N_CODES), lambda i: (0, 0)),
        ],
        out_specs=[
            pl.BlockSpec((BLOCK_ROWS, 1), lambda i: (i, 0)),
            pl.BlockSpec((1, 1), lambda i: (0, 0)),
            pl.BlockSpec((1, 1), lambda i: (0, 0)),
        ],
        out_shape=[
            jax.ShapeDtypeStruct((N_ROWS, 1), jnp.int32),
            jax.ShapeDtypeStruct((1, 1), jnp.float32),
            jax.ShapeDtypeStruct((1, 1), jnp.float32),
        ],
        scratch_shapes=[
            pltpu.SMEM((1, 1), jnp.float32),
        ],
    )(x_flat, xn, en, et, jcol)

    idx3 = idx.reshape(SC_TILES, N_CHUNKS, IDX_CHUNK)
    zeros2d = jnp.zeros((N_CODES, HIST_W), jnp.float32)
    ones2d = jnp.ones((IDX_CHUNK, HIST_W), jnp.float32)
    q, counts = _sc_gather(embedding, idx3, zeros2d, ones2d)

    q_st, pp = pl.pallas_call(
        _st_kernel,
        grid=(N_BLOCKS,),
        in_specs=[
            pl.BlockSpec((BLOCK_ROWS, DIM), lambda i: (i, 0)),
            pl.BlockSpec((BLOCK_ROWS, DIM), lambda i: (i, 0)),
            pl.BlockSpec((SC_CORES, N_CODES, HIST_W), lambda i: (0, 0, 0)),
        ],
        out_specs=[
            pl.BlockSpec((BLOCK_ROWS, DIM), lambda i: (i, 0)),
            pl.BlockSpec((1, 1), lambda i: (0, 0)),
        ],
        out_shape=[
            jax.ShapeDtypeStruct((N_ROWS, DIM), jnp.float32),
            jax.ShapeDtypeStruct((1, 1), jnp.float32),
        ],
    )(x_flat, q, counts)

    quantized_st = q_st.reshape(x.shape)
    return (quantized_st, cm.reshape(()), cb.reshape(()), pp.reshape(()))
